# Initial kernel scaffold; baseline (speedup 1.0000x reference)
import jax
import jax.numpy as jnp
from jax.experimental import pallas as pl


def kernel(x, batch, node_weight, weight, bias, mean_scale):
    raise NotImplementedError("write your pallas kernel here")


# speedup vs baseline: 2.3107x; 2.3107x over previous
import functools

import jax
import jax.numpy as jnp
from jax import lax
from jax.experimental import pallas as pl
from jax.experimental.pallas import tpu as pltpu
from jax.experimental.pallas import tpu_sc as plsc

EPS_K = 1e-05
N_K = 100000
C_K = 128
G_K = 64
NC_K = 2
NS_K = 16
NW_K = NC_K * NS_K
CH_K = 200
NCH_K = N_K // CH_K
CHP_K = CH_K + 16
KV_K = C_K // 16

_MESH = plsc.VectorSubcoreMesh(core_axis_name="c", subcore_axis_name="s")


@functools.partial(
    pl.kernel,
    out_type=[
        jax.ShapeDtypeStruct((NC_K, G_K, C_K), jnp.float32),
        jax.ShapeDtypeStruct((NC_K, G_K, C_K), jnp.float32),
        jax.ShapeDtypeStruct((NC_K, G_K, 16), jnp.float32),
    ],
    mesh=_MESH,
    scratch_types=[
        pltpu.VMEM((CH_K, C_K), jnp.float32),
        pltpu.VMEM((CHP_K,), jnp.int32),
        pltpu.VMEM((CHP_K,), jnp.float32),
        pltpu.VMEM((G_K, C_K), jnp.float32),
        pltpu.VMEM((G_K, C_K), jnp.float32),
        pltpu.VMEM((G_K, 16), jnp.float32),
        pltpu.VMEM((G_K,), jnp.int32),
        pltpu.VMEM_SHARED((G_K, C_K), jnp.float32),
        pltpu.VMEM_SHARED((G_K, C_K), jnp.float32),
        pltpu.VMEM_SHARED((G_K, 16), jnp.float32),
    ],
)
def _sums_kernel(x_hbm, g_hbm, w_hbm, owx, owx2, ow,
                 xbuf, gbuf, wbuf, awx, awx2, aw, idx,
                 swx, swx2, sw):
    c = lax.axis_index("c")
    s = lax.axis_index("s")
    wid = s * NC_K + c
    zero = jnp.zeros((16,), jnp.float32)

    def zbody(i, carry):
        for k in range(KV_K):
            awx[i, pl.ds(k * 16, 16)] = zero
            awx2[i, pl.ds(k * 16, 16)] = zero
        aw[i, :] = zero
        return carry

    lax.fori_loop(0, G_K, zbody, 0)

    iota16 = lax.iota(jnp.int32, 16)
    for i in range(G_K // 16):
        idx[pl.ds(i * 16, 16)] = iota16 + (i * 16)

    @pl.when(s == 0)
    def _init_shared():
        pltpu.sync_copy(awx, swx)
        pltpu.sync_copy(awx2, swx2)
        pltpu.sync_copy(aw, sw)

    plsc.subcore_barrier()

    n_my = (NCH_K - wid + NW_K - 1) // NW_K

    def chunk_body(i, carry):
        row0 = (wid + i * NW_K) * CH_K
        pltpu.sync_copy(x_hbm.at[pl.ds(row0, CH_K)], xbuf)
        pltpu.sync_copy(g_hbm.at[pl.ds(row0, CH_K)], gbuf.at[pl.ds(0, CH_K)])
        pltpu.sync_copy(w_hbm.at[pl.ds(row0, CH_K)], wbuf.at[pl.ds(0, CH_K)])

        def row_body(r, rcarry):
            g = gbuf[pl.ds(r, 16)][0]
            wr = wbuf[pl.ds(r, 16)][0]
            aw[g, :] = aw[g, :] + wr
            for k in range(KV_K):
                sl = pl.ds(k * 16, 16)
                xv = xbuf[r, sl]
                wx = xv * wr
                awx[g, sl] = awx[g, sl] + wx
                awx2[g, sl] = awx2[g, sl] + wx * xv
            return rcarry

        lax.fori_loop(0, CH_K, row_body, 0)
        return carry

    lax.fori_loop(0, n_my, chunk_body, 0)

    pltpu.sync_copy(awx, swx.at[idx], add=True)
    pltpu.sync_copy(awx2, swx2.at[idx], add=True)
    pltpu.sync_copy(aw, sw.at[idx], add=True)
    plsc.subcore_barrier()

    @pl.when(s == 0)
    def _export():
        pltpu.sync_copy(swx, owx.at[c])
        pltpu.sync_copy(swx2, owx2.at[c])
        pltpu.sync_copy(sw, ow.at[c])


@functools.partial(
    pl.kernel,
    out_type=jax.ShapeDtypeStruct((N_K, C_K), jnp.float32),
    mesh=_MESH,
    scratch_types=[
        pltpu.VMEM((CH_K, C_K), jnp.float32),
        pltpu.VMEM((CH_K, C_K), jnp.float32),
        pltpu.VMEM((CHP_K,), jnp.int32),
        pltpu.VMEM((G_K, C_K), jnp.float32),
        pltpu.VMEM((G_K, C_K), jnp.float32),
    ],
)
def _apply_kernel(x_hbm, g_hbm, a_hbm, b_hbm, out_hbm,
                  xbuf, ybuf, gbuf, abuf, obuf):
    c = lax.axis_index("c")
    s = lax.axis_index("s")
    wid = s * NC_K + c

    pltpu.sync_copy(a_hbm, abuf)
    pltpu.sync_copy(b_hbm, obuf)

    n_my = (NCH_K - wid + NW_K - 1) // NW_K

    def chunk_body(i, carry):
        row0 = (wid + i * NW_K) * CH_K
        pltpu.sync_copy(x_hbm.at[pl.ds(row0, CH_K)], xbuf)
        pltpu.sync_copy(g_hbm.at[pl.ds(row0, CH_K)], gbuf.at[pl.ds(0, CH_K)])

        def row_body(r, rcarry):
            g = gbuf[pl.ds(r, 16)][0]
            for k in range(KV_K):
                sl = pl.ds(k * 16, 16)
                ybuf[r, sl] = abuf[g, sl] * xbuf[r, sl] + obuf[g, sl]
            return rcarry

        lax.fori_loop(0, CH_K, row_body, 0)
        pltpu.sync_copy(ybuf, out_hbm.at[pl.ds(row0, CH_K)])
        return carry

    lax.fori_loop(0, n_my, chunk_body, 0)


def kernel(x, batch, node_weight, weight, bias, mean_scale):
    g1 = batch.astype(jnp.int32)

    swx, swx2, sw = _sums_kernel(x, g1, node_weight)
    swx = swx[0] + swx[1]
    swx2 = swx2[0] + swx2[1]
    sumw = (sw[0, :, 0] + sw[1, :, 0])[:, None]

    mean = swx / sumw
    ex2 = swx2 / sumw
    mm = mean * mean_scale[None, :]
    var = ex2 - (2.0 * mm) * mean + mm * mm
    a = weight[None, :] / jnp.sqrt(var + EPS_K)
    b = bias[None, :] - a * mm

    return _apply_kernel(x, g1, a, b)


# speedup vs baseline: 11.1799x; 4.8383x over previous
import functools

import jax
import jax.numpy as jnp
import numpy as np
from jax import lax
from jax.experimental import pallas as pl
from jax.experimental.pallas import tpu as pltpu
from jax.experimental.pallas import tpu_sc as plsc

EPS_K = 1e-05
N_K = 100000
C_K = 128
G_K = 64
NC_K = 2
NS_K = 16
NW_K = NC_K * NS_K
CH_K = 160
NG_K = CH_K // 16
NCH_K = N_K // CH_K
NIT_K = (NCH_K + NW_K - 1) // NW_K
NPAIR_K = NIT_K // 2
KV_K = C_K // 16

_MESH = plsc.VectorSubcoreMesh(core_axis_name="c", subcore_axis_name="s")

def _clamped_chunk(wid, i):
    cid = wid + i * NW_K
    valid = cid < NCH_K
    return jnp.where(valid, cid, 0), valid


@functools.partial(
    pl.kernel,
    out_type=[
        jax.ShapeDtypeStruct((NC_K, G_K, C_K), jnp.float32),
        jax.ShapeDtypeStruct((NC_K, G_K, C_K), jnp.float32),
        jax.ShapeDtypeStruct((NC_K, G_K, 16), jnp.float32),
    ],
    mesh=_MESH,
    scratch_types=[
        pltpu.VMEM((CH_K, C_K), jnp.float32),
        pltpu.VMEM((CH_K, C_K), jnp.float32),
        pltpu.VMEM((CH_K,), jnp.int32),
        pltpu.VMEM((CH_K,), jnp.int32),
        pltpu.VMEM((CH_K,), jnp.float32),
        pltpu.VMEM((CH_K,), jnp.float32),
        pltpu.VMEM((G_K, C_K), jnp.float32),
        pltpu.VMEM((G_K, C_K), jnp.float32),
        pltpu.VMEM((G_K, 16), jnp.float32),
        pltpu.VMEM((G_K,), jnp.int32),
        pltpu.VMEM_SHARED((G_K, C_K), jnp.float32),
        pltpu.VMEM_SHARED((G_K, C_K), jnp.float32),
        pltpu.VMEM_SHARED((G_K, 16), jnp.float32),
        pltpu.SemaphoreType.DMA,
        pltpu.SemaphoreType.DMA,
        pltpu.SemaphoreType.DMA,
        pltpu.SemaphoreType.DMA,
        pltpu.SemaphoreType.DMA,
        pltpu.SemaphoreType.DMA,
    ],
)
def _sums_kernel(x_hbm, g_hbm, w_hbm, owx, owx2, ow,
                 x0, x1, g0, g1, w0, w1, awx, awx2, aw, idx,
                 swx, swx2, sw,
                 sx0, sx1, sg0, sg1, sw0, sw1):
    c = lax.axis_index("c")
    s = lax.axis_index("s")
    wid = s * NC_K + c
    zero = jnp.zeros((16,), jnp.float32)
    xb, gb, wb = [x0, x1], [g0, g1], [w0, w1]
    sx, sg, swm = [sx0, sx1], [sg0, sg1], [sw0, sw1]

    def issue(b, cid):
        row0 = cid * CH_K
        pltpu.async_copy(x_hbm.at[pl.ds(row0, CH_K)], xb[b], sx[b])
        pltpu.async_copy(g_hbm.at[pl.ds(row0, CH_K)], gb[b], sg[b])
        pltpu.async_copy(w_hbm.at[pl.ds(row0, CH_K)], wb[b], swm[b])

    def drain(b, cid):
        row0 = cid * CH_K
        pltpu.make_async_copy(x_hbm.at[pl.ds(row0, CH_K)], xb[b], sx[b]).wait()
        pltpu.make_async_copy(g_hbm.at[pl.ds(row0, CH_K)], gb[b], sg[b]).wait()
        pltpu.make_async_copy(w_hbm.at[pl.ds(row0, CH_K)], wb[b], swm[b]).wait()

    for b in range(2):
        cid, _ = _clamped_chunk(wid, jnp.int32(b))
        issue(b, cid)

    def zbody(i, carry):
        for k in range(KV_K):
            awx[i, pl.ds(k * 16, 16)] = zero
            awx2[i, pl.ds(k * 16, 16)] = zero
        aw[i, :] = zero
        return carry

    lax.fori_loop(0, G_K, zbody, 0)

    iota16 = lax.iota(jnp.int32, 16)
    for i in range(G_K // 16):
        idx[pl.ds(i * 16, 16)] = iota16 + (i * 16)

    @pl.when(s == 0)
    def _init_shared():
        pltpu.sync_copy(awx, swx)
        pltpu.sync_copy(awx2, swx2)
        pltpu.sync_copy(aw, sw)

    plsc.subcore_barrier()

    def run_chunk(xc, gc, wc):
        def group_body(grp, carry):
            r0 = grp * 16
            gv = gc[pl.ds(r0, 16)]
            wv = wc[pl.ds(r0, 16)]

            def fast():
                g = gv[0]
                wacc = wv
                acc = [zero] * KV_K
                acc2 = [zero] * KV_K
                for j in range(16):
                    wr = wv[j]
                    for k in range(KV_K):
                        xv = xc[r0 + j, pl.ds(k * 16, 16)]
                        wx = xv * wr
                        acc[k] = acc[k] + wx
                        acc2[k] = acc2[k] + wx * xv
                aw[g, :] = aw[g, :] + wacc
                for k in range(KV_K):
                    sl = pl.ds(k * 16, 16)
                    awx[g, sl] = awx[g, sl] + acc[k]
                    awx2[g, sl] = awx2[g, sl] + acc2[k]

            def slow():
                for j in range(16):
                    g = gv[j]
                    wr = wv[j]
                    aw[g, :] = aw[g, :] + jnp.where(iota16 == j, wr,
                                                    jnp.float32(0.0))
                    for k in range(KV_K):
                        sl = pl.ds(k * 16, 16)
                        xv = xc[r0 + j, sl]
                        wx = xv * wr
                        awx[g, sl] = awx[g, sl] + wx
                        awx2[g, sl] = awx2[g, sl] + wx * xv

            lax.cond(gv[0] == gv[15], fast, slow)
            return carry

        lax.fori_loop(0, NG_K, group_body, 0)

    def outer(o, carry):
        for b in range(2):
            i = o * 2 + b
            cid, valid = _clamped_chunk(wid, i)
            drain(b, cid)

            @pl.when(valid)
            def _process():
                run_chunk(xb[b], gb[b], wb[b])

            @pl.when(i + 2 < NIT_K)
            def _prefetch():
                cid2, _ = _clamped_chunk(wid, i + 2)
                issue(b, cid2)

        return carry

    lax.fori_loop(0, NPAIR_K, outer, 0)

    pltpu.sync_copy(awx, swx.at[idx], add=True)
    pltpu.sync_copy(awx2, swx2.at[idx], add=True)
    pltpu.sync_copy(aw, sw.at[idx], add=True)
    plsc.subcore_barrier()

    @pl.when(s == 0)
    def _export():
        pltpu.sync_copy(swx, owx.at[c])
        pltpu.sync_copy(swx2, owx2.at[c])
        pltpu.sync_copy(sw, ow.at[c])


@functools.partial(
    pl.kernel,
    out_type=jax.ShapeDtypeStruct((N_K, C_K), jnp.float32),
    mesh=_MESH,
    scratch_types=[
        pltpu.VMEM((CH_K, C_K), jnp.float32),
        pltpu.VMEM((CH_K, C_K), jnp.float32),
        pltpu.VMEM((CH_K, C_K), jnp.float32),
        pltpu.VMEM((CH_K, C_K), jnp.float32),
        pltpu.VMEM((CH_K,), jnp.int32),
        pltpu.VMEM((CH_K,), jnp.int32),
        pltpu.VMEM((G_K, C_K), jnp.float32),
        pltpu.VMEM((G_K, C_K), jnp.float32),
        pltpu.SemaphoreType.DMA,
        pltpu.SemaphoreType.DMA,
        pltpu.SemaphoreType.DMA,
        pltpu.SemaphoreType.DMA,
        pltpu.SemaphoreType.DMA,
        pltpu.SemaphoreType.DMA,
    ],
)
def _apply_kernel(x_hbm, g_hbm, a_hbm, b_hbm, out_hbm,
                  x0, x1, y0, y1, g0, g1, abuf, obuf,
                  sx0, sx1, sy0, sy1, sg0, sg1):
    c = lax.axis_index("c")
    s = lax.axis_index("s")
    wid = s * NC_K + c
    xb, yb, gb = [x0, x1], [y0, y1], [g0, g1]
    sx, sy, sg = [sx0, sx1], [sy0, sy1], [sg0, sg1]

    def issue(b, cid):
        row0 = cid * CH_K
        pltpu.async_copy(x_hbm.at[pl.ds(row0, CH_K)], xb[b], sx[b])
        pltpu.async_copy(g_hbm.at[pl.ds(row0, CH_K)], gb[b], sg[b])

    def drain(b, cid):
        row0 = cid * CH_K
        pltpu.make_async_copy(x_hbm.at[pl.ds(row0, CH_K)], xb[b], sx[b]).wait()
        pltpu.make_async_copy(g_hbm.at[pl.ds(row0, CH_K)], gb[b], sg[b]).wait()

    for b in range(2):
        cid, _ = _clamped_chunk(wid, jnp.int32(b))
        issue(b, cid)

    pltpu.sync_copy(a_hbm, abuf)
    pltpu.sync_copy(b_hbm, obuf)

    def run_chunk(xc, gc, yc):
        def group_body(grp, carry):
            r0 = grp * 16
            gv = gc[pl.ds(r0, 16)]

            def fast():
                g = gv[0]
                av = [abuf[g, pl.ds(k * 16, 16)] for k in range(KV_K)]
                bv = [obuf[g, pl.ds(k * 16, 16)] for k in range(KV_K)]
                for j in range(16):
                    for k in range(KV_K):
                        sl = pl.ds(k * 16, 16)
                        yc[r0 + j, sl] = av[k] * xc[r0 + j, sl] + bv[k]

            def slow():
                for j in range(16):
                    g = gv[j]
                    for k in range(KV_K):
                        sl = pl.ds(k * 16, 16)
                        yc[r0 + j, sl] = abuf[g, sl] * xc[r0 + j, sl] + obuf[g, sl]

            lax.cond(gv[0] == gv[15], fast, slow)
            return carry

        lax.fori_loop(0, NG_K, group_body, 0)

    def outer(o, carry):
        for b in range(2):
            i = o * 2 + b
            cid, valid = _clamped_chunk(wid, i)
            row0 = cid * CH_K
            drain(b, cid)

            @pl.when(i >= 2)
            def _reuse():
                pltpu.make_async_copy(yb[b], out_hbm.at[pl.ds(row0, CH_K)],
                                      sy[b]).wait()

            run_chunk(xb[b], gb[b], yb[b])
            pltpu.async_copy(yb[b], out_hbm.at[pl.ds(row0, CH_K)], sy[b])

            @pl.when(i + 2 < NIT_K)
            def _prefetch():
                cid2, _ = _clamped_chunk(wid, i + 2)
                issue(b, cid2)

        return carry

    lax.fori_loop(0, NPAIR_K, outer, 0)

    for b in range(2):
        cid, _ = _clamped_chunk(wid, NIT_K - 2 + b)
        row0 = cid * CH_K
        pltpu.make_async_copy(yb[b], out_hbm.at[pl.ds(row0, CH_K)], sy[b]).wait()


def kernel(x, batch, node_weight, weight, bias, mean_scale):
    g1 = batch.astype(jnp.int32)

    swx, swx2, sw = _sums_kernel(x, g1, node_weight)
    swx = swx[0] + swx[1]
    swx2 = swx2[0] + swx2[1]
    sumw = (sw[0] + sw[1]).sum(-1)[:, None]

    mean = swx / sumw
    ex2 = swx2 / sumw
    mm = mean * mean_scale[None, :]
    var = ex2 - (2.0 * mm) * mean + mm * mm
    a = weight[None, :] / jnp.sqrt(var + EPS_K)
    b = bias[None, :] - a * mm

    return _apply_kernel(x, g1, a, b)
